# baseline (speedup 1.0000x reference)
import functools

import jax
import jax.numpy as jnp
from jax import lax
from jax.experimental import pallas as pl
from jax.experimental.pallas import tpu as pltpu
from jax.experimental.pallas import tpu_sc as plsc

B = 16
H = 128
ACT_DIM = 4096
T = 32768
N_NODES = 65536

NC = 2
NS = 16
NW = NC * NS
LANES = 16

NSLICE = 4
T2 = T // NSLICE


GATHER_BATCH = 128


@functools.cache
def _make_sc_gather(nrows, row0):
    rows_per_w = nrows // NW
    n_batch = rows_per_w // GATHER_BATCH
    mesh = plsc.VectorSubcoreMesh(
        core_axis_name="c", subcore_axis_name="s", num_cores=NC, num_subcores=NS)

    def body(table_hbm, idx_hbm, out_hbm, idx_v, rows_a, rows_b, sem_a, sem_b):
        wid = lax.axis_index("s") * NC + lax.axis_index("c")
        pltpu.sync_copy(idx_hbm.at[pl.ds(row0 + wid * n_batch, n_batch)], idx_v)
        bufs = (rows_a, rows_b)
        sems = (sem_a, sem_b)
        cps = [None, None]
        cps[0] = pltpu.async_copy(table_hbm.at[idx_v.at[0]], bufs[0], sems[0])
        for j in range(n_batch):
            if j + 1 < n_batch:
                cps[(j + 1) % 2] = pltpu.async_copy(
                    table_hbm.at[idx_v.at[j + 1]], bufs[(j + 1) % 2],
                    sems[(j + 1) % 2])
            cps[j % 2].wait()
            pltpu.sync_copy(
                bufs[j % 2],
                out_hbm.at[pl.ds(wid * rows_per_w + j * GATHER_BATCH,
                                 GATHER_BATCH)])

    return functools.partial(
        pl.kernel,
        out_type=jax.ShapeDtypeStruct((nrows, H), jnp.float32),
        mesh=mesh,
        compiler_params=pltpu.CompilerParams(needs_layout_passes=False),
        scratch_types=[
            pltpu.VMEM((n_batch, GATHER_BATCH), jnp.int32),
            pltpu.VMEM((GATHER_BATCH, H), jnp.float32),
            pltpu.VMEM((GATHER_BATCH, H), jnp.float32),
            pltpu.SemaphoreType.DMA,
            pltpu.SemaphoreType.DMA,
        ],
    )(body)


TB = 4096
NB2 = T2 // TB


def _tc_mlp_body(ev_ref, seg_ref, pooled_ref, w1_ref, b1_ref, w2a_ref, w2b_ref,
                 b2_ref, w3_ref, b3_ref, out_ref):
    ctx = jax.nn.relu(
        jnp.dot(pooled_ref[...], w1_ref[...], preferred_element_type=jnp.float32)
        + b1_ref[...])
    ctx2 = jnp.dot(ctx, w2b_ref[...], preferred_element_type=jnp.float32) + b2_ref[...]
    seg = seg_ref[...].reshape(1, TB)
    onehot_t = (lax.broadcasted_iota(jnp.int32, (B, TB), 0) == seg
                ).astype(jnp.float32)
    ctxg = lax.dot_general(onehot_t, ctx2, (((0,), (0,)), ((), ())),
                           preferred_element_type=jnp.float32)
    u = jnp.dot(ev_ref[...], w2a_ref[...], preferred_element_type=jnp.float32)
    h = jax.nn.relu(u + ctxg)
    vt = lax.dot_general(w3_ref[...], h, (((0,), (1,)), ((), ())),
                         preferred_element_type=jnp.float32)
    out_ref[...] = jnp.tanh(vt + b3_ref[...])


def _tc_mlp(ev_emb, seg3, pooled2, W1, b1, W2a, W2b, b2, W3, b3):
    return pl.pallas_call(
        _tc_mlp_body,
        grid=(NB2,),
        in_specs=[
            pl.BlockSpec((TB, H), lambda i: (i, 0)),
            pl.BlockSpec((1, 1, TB), lambda i: (i, 0, 0)),
            pl.BlockSpec((B, H), lambda i: (0, 0)),
            pl.BlockSpec((H, H), lambda i: (0, 0)),
            pl.BlockSpec((1, H), lambda i: (0, 0)),
            pl.BlockSpec((H, H), lambda i: (0, 0)),
            pl.BlockSpec((H, H), lambda i: (0, 0)),
            pl.BlockSpec((1, H), lambda i: (0, 0)),
            pl.BlockSpec((H, 1), lambda i: (0, 0)),
            pl.BlockSpec((1, 1), lambda i: (0, 0)),
        ],
        out_specs=pl.BlockSpec((1, TB), lambda i: (0, i)),
        out_shape=jax.ShapeDtypeStruct((1, T2), jnp.float32),
    )(ev_emb, seg3, pooled2, W1, b1, W2a, W2b, b2, W3, b3)


NBF = T // TB


def _tc_offs_body(seg_ref, offs_ref):
    acc = jnp.zeros((B, 1), jnp.float32)
    for r in range(NBF):
        seg = seg_ref[r].reshape(1, TB)
        onehot_t = (lax.broadcasted_iota(jnp.int32, (B, TB), 0) == seg
                    ).astype(jnp.float32)
        acc = acc + jnp.sum(onehot_t, axis=1, keepdims=True)
    tri = (lax.broadcasted_iota(jnp.int32, (B, B), 0)
           > lax.broadcasted_iota(jnp.int32, (B, B), 1)).astype(jnp.float32)
    starts = jnp.dot(tri, acc, preferred_element_type=jnp.float32,
                     precision=lax.Precision.HIGHEST)
    ends = starts + acc
    z = jnp.zeros((NW - B, 1), jnp.float32)
    starts_p = jnp.concatenate([starts, z], axis=0)
    ends_p = jnp.concatenate([ends, z], axis=0)
    col = lax.broadcasted_iota(jnp.int32, (NW, LANES), 1)
    out2 = jnp.where(col == 0, starts_p, jnp.where(col == 1, ends_p, 0.0))
    offs_ref[...] = out2.astype(jnp.int32)


def _tc_offs(seg3):
    return pl.pallas_call(
        _tc_offs_body,
        grid=(1,),
        in_specs=[pl.BlockSpec((NBF, 1, TB), lambda i: (0, 0, 0))],
        out_specs=pl.BlockSpec((NW, LANES), lambda i: (0, 0)),
        out_shape=jax.ShapeDtypeStruct((NW, LANES), jnp.int32),
    )(seg3)


CH = 2048
PAD = 2 * CH


@functools.cache
def _make_sc_scatter():
    mesh = plsc.VectorSubcoreMesh(
        core_axis_name="c", subcore_axis_name="s", num_cores=NC, num_subcores=NS)
    return functools.partial(
        pl.kernel,
        out_type=jax.ShapeDtypeStruct((B, ACT_DIM), jnp.float32),
        mesh=mesh,
        compiler_params=pltpu.CompilerParams(needs_layout_passes=False),
        scratch_types=[
            pltpu.VMEM((LANES,), jnp.int32),
            pltpu.VMEM((CH + LANES,), jnp.int32),
            pltpu.VMEM((CH,), jnp.float32),
            pltpu.VMEM((ACT_DIM,), jnp.float32),
        ],
    )(_sc_scatter_body)


def _sc_scatter_body(vals_hbm, map_hbm, offs_hbm, out_hbm, offs_v, idx_v, vals_v, row_v):
    wid = lax.axis_index("s") * NC + lax.axis_index("c")

    @pl.when(wid < B)
    def _work():
        s = wid
        pltpu.sync_copy(offs_hbm.at[s], offs_v)
        o = offs_v[pl.ds(0, LANES)]
        start = o[0]
        end = o[1]
        lane = lax.broadcasted_iota(jnp.int32, (LANES,), 0)
        abase = (start // 8) * 8
        nch = (end - abase + CH - 1) // CH

        zero16 = jnp.zeros((LANES,), jnp.float32)

        def _zero(i, carry):
            row_v[pl.ds(i * LANES, LANES)] = zero16
            return carry

        lax.fori_loop(0, ACT_DIM // LANES, _zero, 0)

        def _chunk(c, carry):
            cb = abase + c * CH
            pltpu.sync_copy(map_hbm.at[pl.ds(cb, CH + LANES)], idx_v)
            pltpu.sync_copy(vals_hbm.at[pl.ds(cb, CH)], vals_v)

            def _group(g, carry2):
                base = g * LANES
                pos = cb + base + lane
                idx16 = idx_v[pl.ds(base, LANES)]
                val16 = vals_v[pl.ds(base, LANES)]
                valid = (pos >= start) & (pos < end)
                _, lastmask = plsc.scan_count(idx16, mask=valid)
                plsc.store_scatter(row_v, [idx16], val16, mask=valid & lastmask)
                return carry2

            lax.fori_loop(0, CH // LANES, _group, 0)
            return carry

        lax.fori_loop(0, nch, _chunk, 0)
        pltpu.sync_copy(row_v, out_hbm.at[s])


def kernel(pooled, node_embeddings, ev_indexes, segment_ids, action_mapper,
           W1, b1, W2, b2, W3, b3):
    pooled2 = pooled[:, 0, :]
    W2a = W2[:H]
    W2b = W2[H:]
    b1r = b1.reshape(1, H)
    b2r = b2.reshape(1, H)
    b3r = b3.reshape(1, 1)
    seg3 = segment_ids.reshape(T // TB, 1, TB)

    offs2 = _tc_offs(seg3)

    idx2d = ev_indexes.reshape(T // GATHER_BATCH, GATHER_BATCH)
    vparts = []
    for si in range(NSLICE):
        gather = _make_sc_gather(T2, si * (T2 // GATHER_BATCH))
        ev_emb = gather(node_embeddings, idx2d)
        seg3_s = lax.slice_in_dim(seg3, si * NB2, (si + 1) * NB2, axis=0)
        vparts.append(_tc_mlp(ev_emb, seg3_s, pooled2, W1, b1r, W2a, W2b, b2r,
                              W3, b3r))

    vals_t = jnp.concatenate(vparts, axis=1)
    vals_pad = jnp.pad(vals_t.reshape(T), (0, PAD))
    map_pad = jnp.pad(action_mapper, (0, PAD))

    return _make_sc_scatter()(vals_pad, map_pad, offs2)


# speedup vs baseline: 1.0966x; 1.0966x over previous
import functools

import jax
import jax.numpy as jnp
from jax import lax
from jax.experimental import pallas as pl
from jax.experimental.pallas import tpu as pltpu
from jax.experimental.pallas import tpu_sc as plsc

B = 16
H = 128
ACT_DIM = 4096
T = 32768
N_NODES = 65536

NC = 2
NS = 16
NW = NC * NS
LANES = 16

NSLICE = 2
T2 = T // NSLICE


GATHER_BATCH = 128


@functools.cache
def _make_sc_gather(nrows, row0):
    rows_per_w = nrows // NW
    n_batch = rows_per_w // GATHER_BATCH
    mesh = plsc.VectorSubcoreMesh(
        core_axis_name="c", subcore_axis_name="s", num_cores=NC, num_subcores=NS)

    def body(table_hbm, idx_hbm, out_hbm, idx_v, rows_a, rows_b, sem_a, sem_b):
        wid = lax.axis_index("s") * NC + lax.axis_index("c")
        pltpu.sync_copy(idx_hbm.at[pl.ds(row0 + wid * n_batch, n_batch)], idx_v)
        bufs = (rows_a, rows_b)
        sems = (sem_a, sem_b)
        cps = [None, None]
        cps[0] = pltpu.async_copy(table_hbm.at[idx_v.at[0]], bufs[0], sems[0])
        for j in range(n_batch):
            if j + 1 < n_batch:
                cps[(j + 1) % 2] = pltpu.async_copy(
                    table_hbm.at[idx_v.at[j + 1]], bufs[(j + 1) % 2],
                    sems[(j + 1) % 2])
            cps[j % 2].wait()
            pltpu.sync_copy(
                bufs[j % 2],
                out_hbm.at[pl.ds(wid * rows_per_w + j * GATHER_BATCH,
                                 GATHER_BATCH)])

    return functools.partial(
        pl.kernel,
        out_type=jax.ShapeDtypeStruct((nrows, H), jnp.float32),
        mesh=mesh,
        compiler_params=pltpu.CompilerParams(needs_layout_passes=False),
        scratch_types=[
            pltpu.VMEM((n_batch, GATHER_BATCH), jnp.int32),
            pltpu.VMEM((GATHER_BATCH, H), jnp.float32),
            pltpu.VMEM((GATHER_BATCH, H), jnp.float32),
            pltpu.SemaphoreType.DMA,
            pltpu.SemaphoreType.DMA,
        ],
    )(body)


TB = 4096
NB2 = T2 // TB


def _tc_mlp_body(ev_ref, seg_ref, pooled_ref, w1_ref, b1_ref, w2a_ref, w2b_ref,
                 b2_ref, w3_ref, b3_ref, out_ref):
    ctx = jax.nn.relu(
        jnp.dot(pooled_ref[...], w1_ref[...], preferred_element_type=jnp.float32)
        + b1_ref[...])
    ctx2 = jnp.dot(ctx, w2b_ref[...], preferred_element_type=jnp.float32) + b2_ref[...]
    seg = seg_ref[...].reshape(1, TB)
    onehot_t = (lax.broadcasted_iota(jnp.int32, (B, TB), 0) == seg
                ).astype(jnp.float32)
    ctxg = lax.dot_general(onehot_t, ctx2, (((0,), (0,)), ((), ())),
                           preferred_element_type=jnp.float32)
    u = jnp.dot(ev_ref[...], w2a_ref[...], preferred_element_type=jnp.float32)
    h = jax.nn.relu(u + ctxg)
    vt = lax.dot_general(w3_ref[...], h, (((0,), (1,)), ((), ())),
                         preferred_element_type=jnp.float32)
    out_ref[...] = jnp.tanh(vt + b3_ref[...])


def _tc_mlp(ev_emb, seg3, pooled2, W1, b1, W2a, W2b, b2, W3, b3):
    return pl.pallas_call(
        _tc_mlp_body,
        grid=(NB2,),
        in_specs=[
            pl.BlockSpec((TB, H), lambda i: (i, 0)),
            pl.BlockSpec((1, 1, TB), lambda i: (i, 0, 0)),
            pl.BlockSpec((B, H), lambda i: (0, 0)),
            pl.BlockSpec((H, H), lambda i: (0, 0)),
            pl.BlockSpec((1, H), lambda i: (0, 0)),
            pl.BlockSpec((H, H), lambda i: (0, 0)),
            pl.BlockSpec((H, H), lambda i: (0, 0)),
            pl.BlockSpec((1, H), lambda i: (0, 0)),
            pl.BlockSpec((H, 1), lambda i: (0, 0)),
            pl.BlockSpec((1, 1), lambda i: (0, 0)),
        ],
        out_specs=pl.BlockSpec((1, TB), lambda i: (0, i)),
        out_shape=jax.ShapeDtypeStruct((1, T2), jnp.float32),
    )(ev_emb, seg3, pooled2, W1, b1, W2a, W2b, b2, W3, b3)


NBF = T // TB


def _tc_offs_body(seg_ref, offs_ref):
    acc = jnp.zeros((B, 1), jnp.float32)
    for r in range(NBF):
        seg = seg_ref[r].reshape(1, TB)
        onehot_t = (lax.broadcasted_iota(jnp.int32, (B, TB), 0) == seg
                    ).astype(jnp.float32)
        acc = acc + jnp.sum(onehot_t, axis=1, keepdims=True)
    tri = (lax.broadcasted_iota(jnp.int32, (B, B), 0)
           > lax.broadcasted_iota(jnp.int32, (B, B), 1)).astype(jnp.float32)
    starts = jnp.dot(tri, acc, preferred_element_type=jnp.float32,
                     precision=lax.Precision.HIGHEST)
    ends = starts + acc
    z = jnp.zeros((NW - B, 1), jnp.float32)
    starts_p = jnp.concatenate([starts, z], axis=0)
    ends_p = jnp.concatenate([ends, z], axis=0)
    col = lax.broadcasted_iota(jnp.int32, (NW, LANES), 1)
    out2 = jnp.where(col == 0, starts_p, jnp.where(col == 1, ends_p, 0.0))
    offs_ref[...] = out2.astype(jnp.int32)


def _tc_offs(seg3):
    return pl.pallas_call(
        _tc_offs_body,
        grid=(1,),
        in_specs=[pl.BlockSpec((NBF, 1, TB), lambda i: (0, 0, 0))],
        out_specs=pl.BlockSpec((NW, LANES), lambda i: (0, 0)),
        out_shape=jax.ShapeDtypeStruct((NW, LANES), jnp.int32),
    )(seg3)


CH = 2048
PAD = 2 * CH


@functools.cache
def _make_sc_scatter():
    mesh = plsc.VectorSubcoreMesh(
        core_axis_name="c", subcore_axis_name="s", num_cores=NC, num_subcores=NS)
    return functools.partial(
        pl.kernel,
        out_type=jax.ShapeDtypeStruct((B, ACT_DIM), jnp.float32),
        mesh=mesh,
        compiler_params=pltpu.CompilerParams(needs_layout_passes=False),
        scratch_types=[
            pltpu.VMEM((LANES,), jnp.int32),
            pltpu.VMEM((CH + LANES,), jnp.int32),
            pltpu.VMEM((CH,), jnp.float32),
            pltpu.VMEM((ACT_DIM,), jnp.float32),
        ],
    )(_sc_scatter_body)


def _sc_scatter_body(vals_hbm, map_hbm, offs_hbm, out_hbm, offs_v, idx_v, vals_v, row_v):
    wid = lax.axis_index("s") * NC + lax.axis_index("c")

    @pl.when(wid < B)
    def _work():
        s = wid
        pltpu.sync_copy(offs_hbm.at[s], offs_v)
        o = offs_v[pl.ds(0, LANES)]
        start = o[0]
        end = o[1]
        lane = lax.broadcasted_iota(jnp.int32, (LANES,), 0)
        abase = (start // 8) * 8
        nch = (end - abase + CH - 1) // CH

        zero16 = jnp.zeros((LANES,), jnp.float32)

        def _zero(i, carry):
            row_v[pl.ds(i * LANES, LANES)] = zero16
            return carry

        lax.fori_loop(0, ACT_DIM // LANES, _zero, 0)

        def _chunk(c, carry):
            cb = abase + c * CH
            pltpu.sync_copy(map_hbm.at[pl.ds(cb, CH + LANES)], idx_v)
            pltpu.sync_copy(vals_hbm.at[pl.ds(cb, CH)], vals_v)

            def _group(g, carry2):
                base = g * LANES
                pos = cb + base + lane
                idx16 = idx_v[pl.ds(base, LANES)]
                val16 = vals_v[pl.ds(base, LANES)]
                valid = (pos >= start) & (pos < end)
                _, lastmask = plsc.scan_count(idx16, mask=valid)
                plsc.store_scatter(row_v, [idx16], val16, mask=valid & lastmask)
                return carry2

            lax.fori_loop(0, CH // LANES, _group, 0)
            return carry

        lax.fori_loop(0, nch, _chunk, 0)
        pltpu.sync_copy(row_v, out_hbm.at[s])


def kernel(pooled, node_embeddings, ev_indexes, segment_ids, action_mapper,
           W1, b1, W2, b2, W3, b3):
    pooled2 = pooled[:, 0, :]
    W2a = W2[:H]
    W2b = W2[H:]
    b1r = b1.reshape(1, H)
    b2r = b2.reshape(1, H)
    b3r = b3.reshape(1, 1)
    seg3 = segment_ids.reshape(T // TB, 1, TB)

    offs2 = _tc_offs(seg3)

    idx2d = ev_indexes.reshape(T // GATHER_BATCH, GATHER_BATCH)
    vparts = []
    for si in range(NSLICE):
        gather = _make_sc_gather(T2, si * (T2 // GATHER_BATCH))
        ev_emb = gather(node_embeddings, idx2d)
        seg3_s = lax.slice_in_dim(seg3, si * NB2, (si + 1) * NB2, axis=0)
        vparts.append(_tc_mlp(ev_emb, seg3_s, pooled2, W1, b1r, W2a, W2b, b2r,
                              W3, b3r))

    vals_t = jnp.concatenate(vparts, axis=1)
    vals_pad = jnp.pad(vals_t.reshape(T), (0, PAD))
    map_pad = jnp.pad(action_mapper, (0, PAD))

    return _make_sc_scatter()(vals_pad, map_pad, offs2)
